# baseline (speedup 1.0000x reference)
import functools

import jax
import jax.numpy as jnp
import numpy as np
from jax import lax
from jax.experimental import pallas as pl
from jax.experimental.pallas import tpu as pltpu
from jax.experimental.pallas import tpu_sc as plsc

N_PTS = 50000
N_PAD = 50176
Q = 2048
K = 8
CHUNK = 25088
N_CHUNKS = N_PAD // CHUNK
SG_VECS = 98
SG_PTS = SG_VECS * 16
SG_UNROLL = 7
P0_VECS = 256
P0_UNROLL = 8
MAXPTR = 512
ROWS = N_PAD // 128
BIG = np.float32(3.0e38)
PAD_LEN = np.float32(1.0e30)
OMC_CLAMP = np.float32(1.0) - (np.float32(1.0) - np.float32(1e-4))

NC, NS = 2, 16
NW = NC * NS
RPW = Q // NW


def _prep_body(dirs_ref, len_ref, rays_ref, out_ref, rout_ref):
    ux = dirs_ref[0]
    uy = dirs_ref[1]
    uz = dirs_ref[2]
    out_ref[0] = ux.astype(jnp.bfloat16).astype(jnp.float32)
    out_ref[1] = uy.astype(jnp.bfloat16).astype(jnp.float32)
    out_ref[2] = uz.astype(jnp.bfloat16).astype(jnp.float32)
    out_ref[3] = len_ref[...]
    out_ref[4] = jnp.arctan2(uy, ux)
    uzc = jnp.clip(uz, -1.0, 1.0)
    out_ref[5] = jnp.arctan2(uzc, jnp.sqrt(jnp.maximum(1.0 - uzc * uzc, 0.0)))
    rout_ref[...] = rays_ref[...].astype(jnp.bfloat16).astype(jnp.float32)


def _prep(dirs_t, len_eff, rays_flat):
    return pl.pallas_call(
        _prep_body,
        out_shape=(
            jax.ShapeDtypeStruct((6, ROWS, 128), jnp.float32),
            jax.ShapeDtypeStruct((48, 128), jnp.float32),
        ),
    )(dirs_t, len_eff, rays_flat)


def _topk_body(pts_hbm, rays_hbm, az_hbm, pi_hbm,
               dist_out, idx_out, az_out, pi_out,
               pbuf, kbuf, vbuf, gaz, gpi, rbuf, dsg, cd, ci, bbuf, tsm, sem):
    w = lax.axis_index("s") * NC + lax.axis_index("c")
    lane = lax.iota(jnp.int32, 16)
    bigv = jnp.full((16,), BIG, jnp.float32)
    zero16 = jnp.zeros((16,), jnp.int32)
    shdn = jnp.maximum(lane - 1, 0)
    lane0 = lane == 0
    seven = jnp.full((16,), 7, jnp.int32)
    xor_idx = [lane ^ c for c in (8, 4, 2, 1)]

    def g16(vec, idx):
        return vec.at[idx].get(mode="promise_in_bounds")

    def min_tree(d):
        m = d
        for idx in xor_idx:
            m = jnp.minimum(m, g16(m, idx))
        return m

    def sum_tree(si):
        for idx in xor_idx:
            si = si + g16(si, idx)
        return si

    def vrow(r):
        return vbuf.at[r // 8, pl.ds((r % 8) * 16, 16)]

    def krow(r):
        return kbuf.at[r // 8, pl.ds((r % 8) * 16, 16)]

    def brow(r):
        return bbuf.at[pl.ds(r * 16, 16)]

    for comp in range(3):
        pltpu.sync_copy(rays_hbm.at[pl.ds(comp * Q + w * RPW, RPW)],
                        rbuf.at[pl.ds(comp * RPW, RPW)])

    def insert_reg(K, V, thr_v, d, iv):
        m = min_tree(d)
        lsel = jnp.where(d == m, lane, 16)
        lmin = min_tree(lsel)
        gidx = g16(iv, lmin)
        cb = jnp.where(m < thr_v, m, bigv)
        Ksh = g16(K, shdn)
        Vsh = g16(V, shdn)
        mk = K <= cb
        msh = jnp.logical_or(Ksh <= cb, lane0)
        K1 = jnp.where(mk, K, jnp.where(msh, cb, Ksh))
        V1 = jnp.where(mk, V, jnp.where(msh, gidx, Vsh))
        thr1 = g16(K1, seven)
        d1 = jnp.where(lane == lmin, bigv, d)
        return K1, V1, thr1, d1

    def process_vec(r, dv, iv, boot_s):
        K = krow(r)[...]
        V = vrow(r)[...]
        thr = g16(K, seven)

        def body(j, carry):
            K, V, thr, d = carry
            return insert_reg(K, V, thr, d, iv)

        K, V, thr, _ = lax.fori_loop(0, 8, body, (K, V, thr, dv))
        krow(r)[...] = K
        vrow(r)[...] = V
        tsm[0] = jnp.minimum(K[7], boot_s)

    def dist_vec(rx, ry, rz, off):
        px = pbuf[0, pl.ds(off, 16)]
        py = pbuf[1, pl.ds(off, 16)]
        pz = pbuf[2, pl.ds(off, 16)]
        plen = pbuf[3, pl.ds(off, 16)]
        cos = rx * px + ry * py + rz * pz
        omc = jnp.where(cos >= 1.0, OMC_CLAMP, 1.0 - cos)
        return omc * plen

    def scan_chunk(c, first):
        pltpu.sync_copy(pts_hbm.at[:, pl.ds(c * CHUNK, CHUNK)], pbuf)

        def ray_body(r, _):
            rsel = jnp.broadcast_to(r % 16, (16,))
            rbase = (r // 16) * 16
            rx = g16(rbuf[pl.ds(rbase, 16)], rsel)
            ry = g16(rbuf[pl.ds(RPW + rbase, 16)], rsel)
            rz = g16(rbuf[pl.ds(2 * RPW + rbase, 16)], rsel)
            if first:
                krow(r)[...] = bigv
                vrow(r)[...] = zero16

                def p0_body(b, M):
                    for j in range(P0_UNROLL):
                        M = jnp.minimum(
                            M, dist_vec(rx, ry, rz, (b * P0_UNROLL + j) * 16))
                    return M

                M = lax.fori_loop(0, P0_VECS // P0_UNROLL, p0_body, bigv)

                def boot_body(j, carry):
                    K, V, thr, d = carry
                    return insert_reg(K, V, thr, d, lane)

                Kb, _, _, _ = lax.fori_loop(
                    0, 8, boot_body, (bigv, zero16, bigv, M))
                boot_s = Kb[7] * np.float32(1.000001) + np.float32(1e-30)
                brow(r)[...] = jnp.minimum(
                    jnp.broadcast_to(boot_s, (16,)), bigv)
                tsm[0] = boot_s
            else:
                boot_s = brow(r)[...][0]
                tsm[0] = jnp.minimum(krow(r)[...][7], boot_s)

            def sg_body(sg, _sg):
                sgoff = sg * SG_PTS

                def fast_body(b, gmin):
                    for j in range(SG_UNROLL):
                        v = b * SG_UNROLL + j
                        dv = dist_vec(rx, ry, rz, sgoff + v * 16)
                        dsg.at[pl.ds(v * 16, 16)][...] = dv
                        gmin = jnp.minimum(gmin, dv)
                    return gmin

                gmin = lax.fori_loop(0, SG_VECS // SG_UNROLL, fast_body, bigv)
                gm = min_tree(gmin)

                @pl.when(gm[0] < tsm[0])
                def _():
                    thr0v = jnp.broadcast_to(tsm[0], (16,))
                    ibase0 = c * CHUNK + sgoff

                    def coll_body(v, ptr):
                        dv = dsg[pl.ds(v * 16, 16)]
                        mask = dv < thr0v
                        dm = jnp.where(mask, dv, bigv)
                        iv = ibase0 + v * 16 + lane
                        cd.at[pl.ds(ptr, 16)][...] = dm
                        ci.at[pl.ds(ptr, 16)][...] = iv
                        adv = jnp.where(min_tree(dm)[0] < BIG, 16, 0)
                        return jnp.minimum(ptr + adv, MAXPTR)

                    ptr = lax.fori_loop(0, SG_VECS, coll_body, 0)
                    cd.at[pl.ds(ptr, 16)][...] = bigv
                    cd.at[pl.ds(ptr + 16, 16)][...] = bigv
                    process_vec(r, cd[pl.ds(0, 16)], ci[pl.ds(0, 16)], boot_s)
                    process_vec(r, cd[pl.ds(16, 16)], ci[pl.ds(16, 16)], boot_s)

                    @pl.when(ptr > 32)
                    def _():
                        for i in range(2, 9):
                            o = jnp.minimum(i * 16, ptr)
                            process_vec(r, cd[pl.ds(o, 16)],
                                        ci[pl.ds(o, 16)], boot_s)

                        @pl.when(ptr > 144)
                        def _():
                            def rest_body(i, _i):
                                o = jnp.minimum(i * 16, ptr)
                                process_vec(r, cd[pl.ds(o, 16)],
                                            ci[pl.ds(o, 16)], boot_s)
                                return 0

                            lax.fori_loop(9, MAXPTR // 16 + 2, rest_body, 0)
                return 0

            lax.fori_loop(0, CHUNK // SG_PTS, sg_body, 0)
            return 0

        lax.fori_loop(0, RPW, ray_body, 0)

    scan_chunk(0, True)
    for c in range(1, N_CHUNKS):
        scan_chunk(c, False)

    for j in range(8):
        pltpu.async_copy(az_hbm.at[vbuf.at[j]], gaz.at[j], sem).wait()
        pltpu.async_copy(pi_hbm.at[vbuf.at[j]], gpi.at[j], sem).wait()

    pltpu.sync_copy(kbuf, dist_out.at[w])
    pltpu.sync_copy(vbuf, idx_out.at[w])
    pltpu.sync_copy(gaz, az_out.at[w])
    pltpu.sync_copy(gpi, pi_out.at[w])


@functools.partial(
    pl.kernel,
    out_type=(
        jax.ShapeDtypeStruct((NW, 8, 128), jnp.float32),
        jax.ShapeDtypeStruct((NW, 8, 128), jnp.int32),
        jax.ShapeDtypeStruct((NW, 8, 128), jnp.float32),
        jax.ShapeDtypeStruct((NW, 8, 128), jnp.float32),
    ),
    mesh=plsc.VectorSubcoreMesh(core_axis_name="c", subcore_axis_name="s"),
    scratch_types=[
        pltpu.VMEM((4, CHUNK), jnp.float32),
        pltpu.VMEM((8, 128), jnp.float32),
        pltpu.VMEM((8, 128), jnp.int32),
        pltpu.VMEM((8, 128), jnp.float32),
        pltpu.VMEM((8, 128), jnp.float32),
        pltpu.VMEM((3 * RPW,), jnp.float32),
        pltpu.VMEM((SG_PTS,), jnp.float32),
        pltpu.VMEM((MAXPTR + 64,), jnp.float32),
        pltpu.VMEM((MAXPTR + 64,), jnp.int32),
        pltpu.VMEM((16 * RPW,), jnp.float32),
        pltpu.SMEM((2,), jnp.float32),
        pltpu.SemaphoreType.DMA,
    ],
)
def _topk_sc(pts_hbm, rays_hbm, az_hbm, pi_hbm,
             dist_out, idx_out, az_out, pi_out,
             pbuf, kbuf, vbuf, gaz, gpi, rbuf, dsg, cd, ci, bbuf, tsm, sem):
    _topk_body(pts_hbm, rays_hbm, az_hbm, pi_hbm,
               dist_out, idx_out, az_out, pi_out,
               pbuf, kbuf, vbuf, gaz, gpi, rbuf, dsg, cd, ci, bbuf, tsm, sem)


def kernel(points, ray_o, ray_d):
    assert points.shape == (N_PTS, 3)
    assert ray_d.shape == (Q, 3)
    ray_d_n = ray_d / jnp.linalg.norm(ray_d, axis=-1, keepdims=True)
    pts_pad = jnp.pad(points, ((0, N_PAD - N_PTS), (0, 0)))
    cam_2_pts = pts_pad - ray_o[None, :]
    direct_len = jnp.linalg.norm(cam_2_pts, axis=1)
    cam_pts_dir = cam_2_pts / direct_len[:, None]

    len_eff = jnp.where(jnp.arange(N_PAD) < N_PTS, direct_len, PAD_LEN)

    out6, rout = _prep(cam_pts_dir.T.reshape(3, ROWS, 128),
                       len_eff.reshape(ROWS, 128),
                       ray_d_n.T.reshape(48, 128))
    prepf = out6.reshape(6, N_PAD)
    pts_soa, az_all, pi_all, rays = lax.optimization_barrier(
        (prepf[:4], prepf[4], prepf[5], rout.reshape(3 * Q)))

    dist, idx, az, pi = _topk_sc(pts_soa, rays, az_all, pi_all)
    dist = dist.reshape(Q, 16)[:, :K]
    idx = idx.reshape(Q, 16)[:, :K]
    az = az.reshape(Q, 16)[:, :K]
    pi = pi.reshape(Q, 16)[:, :K]
    return dist, idx, az, pi


# speedup vs baseline: 1.7233x; 1.7233x over previous
import functools

import jax
import jax.numpy as jnp
import numpy as np
from jax import lax
from jax.experimental import pallas as pl
from jax.experimental.pallas import tpu as pltpu
from jax.experimental.pallas import tpu_sc as plsc

N_PTS = 50000
N_PAD = 50176
Q = 2048
K = 8
CHUNK = 25088
N_CHUNKS = N_PAD // CHUNK
GROUP = 128
NG = CHUNK // GROUP
ROWS = N_PAD // 128
BIG = np.float32(3.0e38)
PAD_LEN = np.float32(1.0e30)
OMC_CLAMP = np.float32(1.0) - (np.float32(1.0) - np.float32(1e-4))

NC, NS = 2, 16
NW = NC * NS
RPW = Q // NW


def _prep_body(dirs_ref, len_ref, rays_ref, out_ref, rout_ref):
    ux = dirs_ref[0]
    uy = dirs_ref[1]
    uz = dirs_ref[2]
    out_ref[0] = ux.astype(jnp.bfloat16).astype(jnp.float32)
    out_ref[1] = uy.astype(jnp.bfloat16).astype(jnp.float32)
    out_ref[2] = uz.astype(jnp.bfloat16).astype(jnp.float32)
    out_ref[3] = len_ref[...]
    out_ref[4] = jnp.arctan2(uy, ux)
    uzc = jnp.clip(uz, -1.0, 1.0)
    out_ref[5] = jnp.arctan2(uzc, jnp.sqrt(jnp.maximum(1.0 - uzc * uzc, 0.0)))
    rout_ref[...] = rays_ref[...].astype(jnp.bfloat16).astype(jnp.float32)


def _prep(dirs_t, len_eff, rays_flat):
    return pl.pallas_call(
        _prep_body,
        out_shape=(
            jax.ShapeDtypeStruct((6, ROWS, 128), jnp.float32),
            jax.ShapeDtypeStruct((48, 128), jnp.float32),
        ),
    )(dirs_t, len_eff, rays_flat)


def _topk_body(pts_hbm, rays_hbm, az_hbm, pi_hbm,
               dist_out, idx_out, az_out, pi_out,
               pbuf, kbuf, vbuf, gaz, gpi, rbuf, tsm, sem):
    w = lax.axis_index("s") * NC + lax.axis_index("c")
    lane = lax.iota(jnp.int32, 16)
    bigv = jnp.full((16,), BIG, jnp.float32)
    shdn = jnp.maximum(lane - 1, 0)
    lane0 = lane == 0
    xor_idx = [lane ^ c for c in (8, 4, 2, 1)]

    def g16(vec, idx):
        return vec.at[idx].get(mode="promise_in_bounds")

    def min_tree(d):
        m = d
        for idx in xor_idx:
            m = jnp.minimum(m, g16(m, idx))
        return m

    def vrow(r):
        return vbuf.at[r // 8, pl.ds((r % 8) * 16, 16)]

    def krow(r):
        return kbuf.at[r // 8, pl.ds((r % 8) * 16, 16)]

    for comp in range(3):
        pltpu.sync_copy(rays_hbm.at[pl.ds(comp * Q + w * RPW, RPW)],
                        rbuf.at[pl.ds(comp * RPW, RPW)])

    def insert(r, d, ibase, guarded):
        m = min_tree(d)
        lsel = jnp.where(d == m, lane, 16)
        lmin = min_tree(lsel)
        gidx = lmin + ibase
        if guarded:
            thr_v = jnp.broadcast_to(tsm[0], (16,))
            cb = jnp.where(m < thr_v, m, bigv)
        else:
            cb = m
        Kv = krow(r)[...]
        Vv = vrow(r)[...]
        Ksh = g16(Kv, shdn)
        Vsh = g16(Vv, shdn)
        mk = Kv <= cb
        msh = jnp.logical_or(Ksh <= cb, lane0)
        K1 = jnp.where(mk, Kv, jnp.where(msh, cb, Ksh))
        V1 = jnp.where(mk, Vv, jnp.where(msh, gidx, Vsh))
        krow(r)[...] = K1
        vrow(r)[...] = V1
        tsm[0] = K1[7]
        return jnp.where(lane == lmin, bigv, d)

    def drain_vec(r, d, ibase):
        m1 = min_tree(d)

        @pl.when(m1[0] < tsm[0])
        def _():
            d1 = insert(r, d, ibase, False)
            m2 = min_tree(d1)

            @pl.when(m2[0] < tsm[0])
            def _():
                def body(j, dj):
                    return insert(r, dj, ibase, True)
                lax.fori_loop(0, 7, body, d1)

    def scan_chunk(c, first):
        pltpu.sync_copy(pts_hbm.at[:, pl.ds(c * CHUNK, CHUNK)], pbuf)

        def ray_body(r, _):
            rsel = jnp.broadcast_to(r % 16, (16,))
            rbase = (r // 16) * 16
            rx = g16(rbuf[pl.ds(rbase, 16)], rsel)
            ry = g16(rbuf[pl.ds(RPW + rbase, 16)], rsel)
            rz = g16(rbuf[pl.ds(2 * RPW + rbase, 16)], rsel)
            if first:
                krow(r)[...] = bigv
                vrow(r)[...] = jnp.zeros((16,), jnp.int32)
                tsm[0] = BIG
            else:
                tsm[0] = krow(r)[...][7]

            def group_body(g, _g):
                off = g * GROUP
                ds = []
                for k in range(GROUP // 16):
                    px = pbuf[0, pl.ds(off + k * 16, 16)]
                    py = pbuf[1, pl.ds(off + k * 16, 16)]
                    pz = pbuf[2, pl.ds(off + k * 16, 16)]
                    plen = pbuf[3, pl.ds(off + k * 16, 16)]
                    cos = rx * px + ry * py + rz * pz
                    omc = jnp.where(cos >= 1.0, OMC_CLAMP, 1.0 - cos)
                    ds.append(omc * plen)
                gmin = ds[0]
                for k in range(1, GROUP // 16):
                    gmin = jnp.minimum(gmin, ds[k])
                gm = min_tree(gmin)

                @pl.when(gm[0] < tsm[0])
                def _():
                    for k in range(GROUP // 16):
                        drain_vec(r, ds[k], c * CHUNK + g * GROUP + k * 16)
                return 0

            lax.fori_loop(0, NG, group_body, 0)
            return 0

        lax.fori_loop(0, RPW, ray_body, 0)

    scan_chunk(0, True)
    for c in range(1, N_CHUNKS):
        scan_chunk(c, False)

    for j in range(8):
        pltpu.async_copy(az_hbm.at[vbuf.at[j]], gaz.at[j], sem).wait()
        pltpu.async_copy(pi_hbm.at[vbuf.at[j]], gpi.at[j], sem).wait()

    pltpu.sync_copy(kbuf, dist_out.at[w])
    pltpu.sync_copy(vbuf, idx_out.at[w])
    pltpu.sync_copy(gaz, az_out.at[w])
    pltpu.sync_copy(gpi, pi_out.at[w])


@functools.partial(
    pl.kernel,
    out_type=(
        jax.ShapeDtypeStruct((NW, 8, 128), jnp.float32),
        jax.ShapeDtypeStruct((NW, 8, 128), jnp.int32),
        jax.ShapeDtypeStruct((NW, 8, 128), jnp.float32),
        jax.ShapeDtypeStruct((NW, 8, 128), jnp.float32),
    ),
    mesh=plsc.VectorSubcoreMesh(core_axis_name="c", subcore_axis_name="s"),
    scratch_types=[
        pltpu.VMEM((4, CHUNK), jnp.float32),
        pltpu.VMEM((8, 128), jnp.float32),
        pltpu.VMEM((8, 128), jnp.int32),
        pltpu.VMEM((8, 128), jnp.float32),
        pltpu.VMEM((8, 128), jnp.float32),
        pltpu.VMEM((3 * RPW,), jnp.float32),
        pltpu.SMEM((2,), jnp.float32),
        pltpu.SemaphoreType.DMA,
    ],
)
def _topk_sc(pts_hbm, rays_hbm, az_hbm, pi_hbm,
             dist_out, idx_out, az_out, pi_out,
             pbuf, kbuf, vbuf, gaz, gpi, rbuf, tsm, sem):
    _topk_body(pts_hbm, rays_hbm, az_hbm, pi_hbm,
               dist_out, idx_out, az_out, pi_out,
               pbuf, kbuf, vbuf, gaz, gpi, rbuf, tsm, sem)


def kernel(points, ray_o, ray_d):
    assert points.shape == (N_PTS, 3)
    assert ray_d.shape == (Q, 3)
    ray_d_n = ray_d / jnp.linalg.norm(ray_d, axis=-1, keepdims=True)
    pts_pad = jnp.pad(points, ((0, N_PAD - N_PTS), (0, 0)))
    cam_2_pts = pts_pad - ray_o[None, :]
    direct_len = jnp.linalg.norm(cam_2_pts, axis=1)
    cam_pts_dir = cam_2_pts / direct_len[:, None]

    len_eff = jnp.where(jnp.arange(N_PAD) < N_PTS, direct_len, PAD_LEN)

    out6, rout = _prep(cam_pts_dir.T.reshape(3, ROWS, 128),
                       len_eff.reshape(ROWS, 128),
                       ray_d_n.T.reshape(48, 128))
    prepf = out6.reshape(6, N_PAD)
    pts_soa, az_all, pi_all, rays = lax.optimization_barrier(
        (prepf[:4], prepf[4], prepf[5], rout.reshape(3 * Q)))

    dist, idx, az, pi = _topk_sc(pts_soa, rays, az_all, pi_all)
    dist = dist.reshape(Q, 16)[:, :K]
    idx = idx.reshape(Q, 16)[:, :K]
    az = az.reshape(Q, 16)[:, :K]
    pi = pi.reshape(Q, 16)[:, :K]
    return dist, idx, az, pi
